# Initial kernel scaffold; baseline (speedup 1.0000x reference)
import functools

import jax
import jax.numpy as jnp
from jax import lax
from jax.experimental import pallas as pl
from jax.experimental.pallas import tpu as pltpu
from jax.experimental.pallas import tpu_sc as plsc

F32 = jnp.float32
W = 128
NC, NS = 2, 16
NW = NC * NS


def _seg_sum_sc(y, src, dst, n_seg):
    E = src.shape[0]
    D = y.shape[1]
    e_per_w = E // NW
    n_chunks = e_per_w // W
    rpt = n_seg // NS
    n_rows = n_seg + 16

    mesh = plsc.VectorSubcoreMesh(core_axis_name="c", subcore_axis_name="s")

    @functools.partial(
        pl.kernel,
        out_type=(jax.ShapeDtypeStruct((NC, n_seg, D), F32),
                  jax.ShapeDtypeStruct((NC, n_seg, 16), F32)),
        mesh=mesh,
        scratch_types=[
            pltpu.VMEM((W,), jnp.int32),
            pltpu.VMEM((W,), jnp.int32),
            pltpu.VMEM((W, D), F32),
            pltpu.VMEM((W, 16), F32),
            pltpu.VMEM_SHARED((n_rows, D), F32),
            pltpu.VMEM_SHARED((n_rows, 16), F32),
            pltpu.SemaphoreType.DMA,
        ],
    )
    def ker(y_hbm, src_hbm, dst_hbm, z_d_hbm, z_c_hbm, ones_hbm,
            acc_out, cnt_out,
            sidx_v, didx_v, rows_v, ones_v, acc_sh, cnt_sh, sem):
        cid = lax.axis_index("c")
        sid = lax.axis_index("s")
        row0 = sid * rpt
        pltpu.sync_copy(z_d_hbm.at[pl.ds(row0, rpt)], acc_sh.at[pl.ds(row0, rpt)])
        pltpu.sync_copy(z_c_hbm.at[pl.ds(row0, rpt)], cnt_sh.at[pl.ds(row0, rpt)])
        pltpu.sync_copy(ones_hbm, ones_v)
        plsc.subcore_barrier()

        ebase = (cid * NS + sid) * e_per_w

        @pl.loop(0, n_chunks)
        def _(k):
            base = ebase + k * W
            pltpu.sync_copy(src_hbm.at[pl.ds(base, W)], sidx_v)
            pltpu.sync_copy(dst_hbm.at[pl.ds(base, W)], didx_v)
            pltpu.async_copy(y_hbm.at[sidx_v], rows_v, sem).wait()
            pltpu.sync_copy(rows_v, acc_sh.at[didx_v], add=True)
            pltpu.sync_copy(ones_v, cnt_sh.at[didx_v], add=True)

        plsc.subcore_barrier()
        pltpu.sync_copy(acc_sh.at[pl.ds(row0, rpt)],
                        acc_out.at[cid, pl.ds(row0, rpt)])
        pltpu.sync_copy(cnt_sh.at[pl.ds(row0, rpt)],
                        cnt_out.at[cid, pl.ds(row0, rpt)])

    zeros_d = jnp.zeros((n_seg, D), F32)
    zeros_c = jnp.zeros((n_seg, 16), F32)
    ones = jnp.ones((W, 16), F32)
    return ker(y, src, dst, zeros_d, zeros_c, ones)


def _pad_edges(src, dst, n_seg, e_pad):
    E = src.shape[0]
    src = jnp.concatenate([src, jnp.zeros((e_pad - E,), src.dtype)])
    dst = jnp.concatenate([dst, jnp.full((e_pad - E,), n_seg, dst.dtype)])
    return src, dst


def _mm_body(x_ref, w_ref, o_ref):
    o_ref[...] = jnp.dot(x_ref[...], w_ref[...], preferred_element_type=F32)


def _project(x, w, bm):
    M, K = x.shape
    N = w.shape[1]
    return pl.pallas_call(
        _mm_body,
        grid=(M // bm,),
        in_specs=[pl.BlockSpec((bm, K), lambda i: (i, 0)),
                  pl.BlockSpec((K, N), lambda i: (0, 0))],
        out_specs=pl.BlockSpec((bm, N), lambda i: (i, 0)),
        out_shape=jax.ShapeDtypeStruct((M, N), F32),
    )(x, w)


def _combine1_body(x_ref, wt_ref, b_ref, a0_ref, a1_ref, c0_ref, c1_ref,
                   wb2_ref, h_ref, z_ref):
    cnt = c0_ref[:, :1] + c1_ref[:, :1]
    agg = (a0_ref[...] + a1_ref[...]) / jnp.maximum(cnt, 1.0)
    h = jnp.dot(x_ref[...], wt_ref[...], preferred_element_type=F32)
    h = jnp.maximum(h + b_ref[...] + agg, 0.0)
    h_ref[...] = h
    z_ref[...] = jnp.dot(h, wb2_ref[...], preferred_element_type=F32)


def _combine1(x, wt, b, a0, a1, c0, c1, wb2, bm):
    M, K = x.shape
    N = wt.shape[1]
    return pl.pallas_call(
        _combine1_body,
        grid=(M // bm,),
        in_specs=[pl.BlockSpec((bm, K), lambda i: (i, 0)),
                  pl.BlockSpec((K, N), lambda i: (0, 0)),
                  pl.BlockSpec((1, N), lambda i: (0, 0)),
                  pl.BlockSpec((bm, N), lambda i: (i, 0)),
                  pl.BlockSpec((bm, N), lambda i: (i, 0)),
                  pl.BlockSpec((bm, 16), lambda i: (i, 0)),
                  pl.BlockSpec((bm, 16), lambda i: (i, 0)),
                  pl.BlockSpec((N, N), lambda i: (0, 0))],
        out_specs=[pl.BlockSpec((bm, N), lambda i: (i, 0)),
                   pl.BlockSpec((bm, N), lambda i: (i, 0))],
        out_shape=[jax.ShapeDtypeStruct((M, N), F32),
                   jax.ShapeDtypeStruct((M, N), F32)],
    )(x, wt, b, a0, a1, c0, c1, wb2)


def _combine2_body(h_ref, wt_ref, b_ref, a0_ref, a1_ref, c0_ref, c1_ref,
                   o_ref):
    cnt = c0_ref[:, :1] + c1_ref[:, :1]
    agg = (a0_ref[...] + a1_ref[...]) / jnp.maximum(cnt, 1.0)
    o_ref[...] = (jnp.dot(h_ref[...], wt_ref[...], preferred_element_type=F32)
                  + b_ref[...] + agg)


def _combine2(h, wt, b, a0, a1, c0, c1):
    M, _ = h.shape
    N = wt.shape[1]
    return pl.pallas_call(
        _combine2_body,
        out_shape=jax.ShapeDtypeStruct((M, N), F32),
    )(h, wt, b, a0, a1, c0, c1)


def kernel(x, src1, dst1, src2, dst2, w1, b1, w2, b2):
    D = 64
    N2, N3 = 10000, 2048
    w1t, w1b = w1[:2 * D], w1[2 * D:]
    w2t, w2b = w2[:D], w2[D:]

    y1 = _project(x, w1b, 2000)
    s1p, d1p = _pad_edges(src1, dst1, N2, 819200)
    acc1, cnt1 = _seg_sum_sc(y1, s1p, d1p, N2)

    h, z2 = _combine1(x[:N2], w1t, b1.reshape(1, -1),
                      acc1[0], acc1[1], cnt1[0], cnt1[1], w2b, 1000)

    s2p, d2p = _pad_edges(src2, dst2, N3, 163840)
    acc2, cnt2 = _seg_sum_sc(z2, s2p, d2p, N3)

    out = _combine2(h[:N3], w2t, b2.reshape(1, -1),
                    acc2[0], acc2[1], cnt2[0], cnt2[1])
    return out


# speedup vs baseline: 5.7521x; 5.7521x over previous
import functools

import jax
import jax.numpy as jnp
from jax import lax
from jax.experimental import pallas as pl
from jax.experimental.pallas import tpu as pltpu
from jax.experimental.pallas import tpu_sc as plsc

F32 = jnp.float32
W = 128
NC, NS = 2, 16
NW = NC * NS


def _seg_sum_sc(y, src, dst, n_pad):
    E = src.shape[0]
    D = y.shape[1]
    e_per_w = E // NW
    n_chunks = e_per_w // W
    rpt = n_pad // NS

    mesh = plsc.VectorSubcoreMesh(core_axis_name="c", subcore_axis_name="s")

    @functools.partial(
        pl.kernel,
        out_type=(jax.ShapeDtypeStruct((NC, n_pad, D), F32),
                  jax.ShapeDtypeStruct((NC, n_pad, 16), F32)),
        mesh=mesh,
        scratch_types=[
            pltpu.VMEM((W,), jnp.int32),
            pltpu.VMEM((W,), jnp.int32),
            pltpu.VMEM((W, D), F32),
            pltpu.VMEM((W, 16), F32),
            pltpu.VMEM_SHARED((n_pad, D), F32),
            pltpu.VMEM_SHARED((n_pad, 16), F32),
            pltpu.SemaphoreType.DMA,
        ],
        compiler_params=pltpu.CompilerParams(use_tc_tiling_on_sc=False),
    )
    def ker(y_hbm, src_hbm, dst_hbm, z_d_hbm, z_c_hbm, ones_hbm,
            acc_out, cnt_out,
            sidx_v, didx_v, rows_v, ones_v, acc_sh, cnt_sh, sem):
        cid = lax.axis_index("c")
        sid = lax.axis_index("s")
        row0 = sid * rpt
        pltpu.sync_copy(z_d_hbm.at[pl.ds(row0, rpt)], acc_sh.at[pl.ds(row0, rpt)])
        pltpu.sync_copy(z_c_hbm.at[pl.ds(row0, rpt)], cnt_sh.at[pl.ds(row0, rpt)])
        pltpu.sync_copy(ones_hbm, ones_v)
        plsc.subcore_barrier()

        ebase = (cid * NS + sid) * e_per_w

        @pl.loop(0, n_chunks)
        def _(k):
            base = ebase + k * W
            pltpu.sync_copy(src_hbm.at[pl.ds(base, W)], sidx_v)
            pltpu.sync_copy(dst_hbm.at[pl.ds(base, W)], didx_v)
            pltpu.async_copy(y_hbm.at[sidx_v], rows_v, sem).wait()
            pltpu.sync_copy(rows_v, acc_sh.at[didx_v], add=True)
            pltpu.sync_copy(ones_v, cnt_sh.at[didx_v], add=True)

        plsc.subcore_barrier()
        pltpu.sync_copy(acc_sh.at[pl.ds(row0, rpt)],
                        acc_out.at[cid, pl.ds(row0, rpt)])
        pltpu.sync_copy(cnt_sh.at[pl.ds(row0, rpt)],
                        cnt_out.at[cid, pl.ds(row0, rpt)])

    zeros_d = jnp.zeros((n_pad, D), F32)
    zeros_c = jnp.zeros((n_pad, 16), F32)
    ones = jnp.ones((W, 16), F32)
    return ker(y, src, dst, zeros_d, zeros_c, ones)


def _pad_edges(src, dst, n_seg, e_pad):
    E = src.shape[0]
    src = jnp.concatenate([src, jnp.zeros((e_pad - E,), src.dtype)])
    dst = jnp.concatenate([dst, jnp.full((e_pad - E,), n_seg, dst.dtype)])
    return src, dst


def _mm_body(x_ref, w_ref, o_ref):
    o_ref[...] = jnp.dot(x_ref[...], w_ref[...], preferred_element_type=F32)


def _project(x, w, bm):
    M, K = x.shape
    N = w.shape[1]
    return pl.pallas_call(
        _mm_body,
        grid=(M // bm,),
        in_specs=[pl.BlockSpec((bm, K), lambda i: (i, 0)),
                  pl.BlockSpec((K, N), lambda i: (0, 0))],
        out_specs=pl.BlockSpec((bm, N), lambda i: (i, 0)),
        out_shape=jax.ShapeDtypeStruct((M, N), F32),
    )(x, w)


def _combine1_body(x_ref, wt_ref, b_ref, a0_ref, a1_ref, c0_ref, c1_ref,
                   wb2_ref, h_ref, z_ref):
    cnt = c0_ref[:, :1] + c1_ref[:, :1]
    agg = (a0_ref[...] + a1_ref[...]) / jnp.maximum(cnt, 1.0)
    h = jnp.dot(x_ref[...], wt_ref[...], preferred_element_type=F32)
    h = jnp.maximum(h + b_ref[...] + agg, 0.0)
    h_ref[...] = h
    z_ref[...] = jnp.dot(h, wb2_ref[...], preferred_element_type=F32)


def _combine1(x, wt, b, a0, a1, c0, c1, wb2, bm):
    M, K = x.shape
    N = wt.shape[1]
    return pl.pallas_call(
        _combine1_body,
        grid=(M // bm,),
        in_specs=[pl.BlockSpec((bm, K), lambda i: (i, 0)),
                  pl.BlockSpec((K, N), lambda i: (0, 0)),
                  pl.BlockSpec((1, N), lambda i: (0, 0)),
                  pl.BlockSpec((bm, N), lambda i: (i, 0)),
                  pl.BlockSpec((bm, N), lambda i: (i, 0)),
                  pl.BlockSpec((bm, 16), lambda i: (i, 0)),
                  pl.BlockSpec((bm, 16), lambda i: (i, 0)),
                  pl.BlockSpec((N, N), lambda i: (0, 0))],
        out_specs=[pl.BlockSpec((bm, N), lambda i: (i, 0)),
                   pl.BlockSpec((bm, N), lambda i: (i, 0))],
        out_shape=[jax.ShapeDtypeStruct((M, N), F32),
                   jax.ShapeDtypeStruct((M, N), F32)],
    )(x, wt, b, a0, a1, c0, c1, wb2)


def _combine2_body(h_ref, wt_ref, b_ref, a0_ref, a1_ref, c0_ref, c1_ref,
                   o_ref):
    cnt = c0_ref[:, :1] + c1_ref[:, :1]
    agg = (a0_ref[...] + a1_ref[...]) / jnp.maximum(cnt, 1.0)
    o_ref[...] = (jnp.dot(h_ref[...], wt_ref[...], preferred_element_type=F32)
                  + b_ref[...] + agg)


def _combine2(h, wt, b, a0, a1, c0, c1):
    M, K = h.shape
    N = wt.shape[1]
    return pl.pallas_call(
        _combine2_body,
        grid=(1,),
        in_specs=[pl.BlockSpec((M, K), lambda i: (0, 0)),
                  pl.BlockSpec((K, N), lambda i: (0, 0)),
                  pl.BlockSpec((1, N), lambda i: (0, 0)),
                  pl.BlockSpec((M, N), lambda i: (0, 0)),
                  pl.BlockSpec((M, N), lambda i: (0, 0)),
                  pl.BlockSpec((M, 16), lambda i: (0, 0)),
                  pl.BlockSpec((M, 16), lambda i: (0, 0))],
        out_specs=pl.BlockSpec((M, N), lambda i: (0, 0)),
        out_shape=jax.ShapeDtypeStruct((M, N), F32),
    )(h, wt, b, a0, a1, c0, c1)


def kernel(x, src1, dst1, src2, dst2, w1, b1, w2, b2):
    D = 64
    N2, N3 = 10000, 2048
    w1t, w1b = w1[:2 * D], w1[2 * D:]
    w2t, w2b = w2[:D], w2[D:]

    y1 = _project(x, w1b, 2000)
    s1p, d1p = _pad_edges(src1, dst1, N2, 819200)
    acc1, cnt1 = _seg_sum_sc(y1, s1p, d1p, 10112)

    h, z2 = _combine1(x[:N2], w1t, b1.reshape(1, -1),
                      acc1[0], acc1[1], cnt1[0], cnt1[1], w2b, 1000)

    s2p, d2p = _pad_edges(src2, dst2, N3, 163840)
    acc2, cnt2 = _seg_sum_sc(z2, s2p, d2p, 2176)

    out = _combine2(h[:N3], w2t, b2.reshape(1, -1),
                    acc2[0], acc2[1], cnt2[0], cnt2[1])
    return out


# speedup vs baseline: 8.6007x; 1.4952x over previous
import functools

import jax
import jax.numpy as jnp
from jax import lax
from jax.experimental import pallas as pl
from jax.experimental.pallas import tpu as pltpu
from jax.experimental.pallas import tpu_sc as plsc

F32 = jnp.float32
W = 128
NC, NS = 2, 16
NW = NC * NS


def _seg_sum_sc(y, src2d, dst2d, n_pad):
    D = y.shape[1]
    total_chunks = src2d.shape[0]
    n_chunks = total_chunks // NW
    rpt = n_pad // NS

    mesh = plsc.VectorSubcoreMesh(core_axis_name="c", subcore_axis_name="s")

    @functools.partial(
        pl.kernel,
        out_type=(jax.ShapeDtypeStruct((NC, n_pad, D), F32),
                  jax.ShapeDtypeStruct((NC, n_pad, 16), F32)),
        mesh=mesh,
        scratch_types=[
            pltpu.VMEM((n_chunks, W), jnp.int32),
            pltpu.VMEM((n_chunks, W), jnp.int32),
            pltpu.VMEM((2, W, D), F32),
            pltpu.VMEM((W, 16), F32),
            pltpu.VMEM_SHARED((n_pad, D), F32),
            pltpu.VMEM_SHARED((n_pad, 16), F32),
            pltpu.SemaphoreType.DMA,
            pltpu.SemaphoreType.DMA,
        ],
        compiler_params=pltpu.CompilerParams(use_tc_tiling_on_sc=False),
    )
    def ker(y_hbm, src_hbm, dst_hbm, z_d_hbm, z_c_hbm, ones_hbm,
            acc_out, cnt_out,
            sidx_v, didx_v, rows_v, ones_v, acc_sh, cnt_sh,
            gsem0, gsem1):
        cid = lax.axis_index("c")
        sid = lax.axis_index("s")
        row0 = sid * rpt
        chunk0 = (cid * NS + sid) * n_chunks

        pltpu.sync_copy(src_hbm.at[pl.ds(chunk0, n_chunks)], sidx_v)
        pltpu.sync_copy(dst_hbm.at[pl.ds(chunk0, n_chunks)], didx_v)
        pltpu.sync_copy(z_d_hbm.at[pl.ds(row0, rpt)], acc_sh.at[pl.ds(row0, rpt)])
        pltpu.sync_copy(z_c_hbm.at[pl.ds(row0, rpt)], cnt_sh.at[pl.ds(row0, rpt)])
        pltpu.sync_copy(ones_hbm, ones_v)
        plsc.subcore_barrier()

        pltpu.async_copy(y_hbm.at[sidx_v.at[0]], rows_v.at[0], gsem0)
        gsems = (gsem0, gsem1)

        @pl.loop(0, n_chunks, step=2)
        def _(k0):
            for b in range(2):
                k = k0 + b

                @pl.when(k + 1 < n_chunks)
                def _():
                    pltpu.async_copy(y_hbm.at[sidx_v.at[k + 1]],
                                     rows_v.at[1 - b], gsems[1 - b])

                pltpu.make_async_copy(y_hbm.at[sidx_v.at[k]],
                                      rows_v.at[b], gsems[b]).wait()
                pltpu.sync_copy(rows_v.at[b], acc_sh.at[didx_v.at[k]],
                                add=True)
                pltpu.sync_copy(ones_v, cnt_sh.at[didx_v.at[k]],
                                add=True)

        plsc.subcore_barrier()
        pltpu.sync_copy(acc_sh.at[pl.ds(row0, rpt)],
                        acc_out.at[cid, pl.ds(row0, rpt)])
        pltpu.sync_copy(cnt_sh.at[pl.ds(row0, rpt)],
                        cnt_out.at[cid, pl.ds(row0, rpt)])

    zeros_d = jnp.zeros((n_pad, D), F32)
    zeros_c = jnp.zeros((n_pad, 16), F32)
    ones = jnp.ones((W, 16), F32)
    return ker(y, src2d, dst2d, zeros_d, zeros_c, ones)


def _pad_edges(src, dst, n_seg, e_pad):
    E = src.shape[0]
    src = jnp.concatenate([src, jnp.zeros((e_pad - E,), src.dtype)])
    dst = jnp.concatenate([dst, jnp.full((e_pad - E,), n_seg, dst.dtype)])
    return src, dst


def _mm_body(x_ref, w_ref, o_ref):
    o_ref[...] = jnp.dot(x_ref[...], w_ref[...], preferred_element_type=F32)


def _project(x, w, bm):
    M, K = x.shape
    N = w.shape[1]
    return pl.pallas_call(
        _mm_body,
        grid=(M // bm,),
        in_specs=[pl.BlockSpec((bm, K), lambda i: (i, 0)),
                  pl.BlockSpec((K, N), lambda i: (0, 0))],
        out_specs=pl.BlockSpec((bm, N), lambda i: (i, 0)),
        out_shape=jax.ShapeDtypeStruct((M, N), F32),
    )(x, w)


def _combine1_body(x_ref, wt_ref, b_ref, a0_ref, a1_ref, c0_ref, c1_ref,
                   wb2_ref, h_ref, z_ref):
    cnt = c0_ref[:, :1] + c1_ref[:, :1]
    agg = (a0_ref[...] + a1_ref[...]) / jnp.maximum(cnt, 1.0)
    h = jnp.dot(x_ref[...], wt_ref[...], preferred_element_type=F32)
    h = jnp.maximum(h + b_ref[...] + agg, 0.0)
    h_ref[...] = h
    z_ref[...] = jnp.dot(h, wb2_ref[...], preferred_element_type=F32)


def _combine1(x, wt, b, a0, a1, c0, c1, wb2, bm):
    M, K = x.shape
    N = wt.shape[1]
    return pl.pallas_call(
        _combine1_body,
        grid=(M // bm,),
        in_specs=[pl.BlockSpec((bm, K), lambda i: (i, 0)),
                  pl.BlockSpec((K, N), lambda i: (0, 0)),
                  pl.BlockSpec((1, N), lambda i: (0, 0)),
                  pl.BlockSpec((bm, N), lambda i: (i, 0)),
                  pl.BlockSpec((bm, N), lambda i: (i, 0)),
                  pl.BlockSpec((bm, 16), lambda i: (i, 0)),
                  pl.BlockSpec((bm, 16), lambda i: (i, 0)),
                  pl.BlockSpec((N, N), lambda i: (0, 0))],
        out_specs=[pl.BlockSpec((bm, N), lambda i: (i, 0)),
                   pl.BlockSpec((bm, N), lambda i: (i, 0))],
        out_shape=[jax.ShapeDtypeStruct((M, N), F32),
                   jax.ShapeDtypeStruct((M, N), F32)],
    )(x, wt, b, a0, a1, c0, c1, wb2)


def _combine2_body(h_ref, wt_ref, b_ref, a0_ref, a1_ref, c0_ref, c1_ref,
                   o_ref):
    cnt = c0_ref[:, :1] + c1_ref[:, :1]
    agg = (a0_ref[...] + a1_ref[...]) / jnp.maximum(cnt, 1.0)
    o_ref[...] = (jnp.dot(h_ref[...], wt_ref[...], preferred_element_type=F32)
                  + b_ref[...] + agg)


def _combine2(h, wt, b, a0, a1, c0, c1):
    M, K = h.shape
    N = wt.shape[1]
    return pl.pallas_call(
        _combine2_body,
        grid=(1,),
        in_specs=[pl.BlockSpec((M, K), lambda i: (0, 0)),
                  pl.BlockSpec((K, N), lambda i: (0, 0)),
                  pl.BlockSpec((1, N), lambda i: (0, 0)),
                  pl.BlockSpec((M, N), lambda i: (0, 0)),
                  pl.BlockSpec((M, N), lambda i: (0, 0)),
                  pl.BlockSpec((M, 16), lambda i: (0, 0)),
                  pl.BlockSpec((M, 16), lambda i: (0, 0))],
        out_specs=pl.BlockSpec((M, N), lambda i: (0, 0)),
        out_shape=jax.ShapeDtypeStruct((M, N), F32),
    )(h, wt, b, a0, a1, c0, c1)


def kernel(x, src1, dst1, src2, dst2, w1, b1, w2, b2):
    D = 64
    N2, N3 = 10000, 2048
    w1t, w1b = w1[:2 * D], w1[2 * D:]
    w2t, w2b = w2[:D], w2[D:]

    y1 = _project(x, w1b, 2000)
    s1p, d1p = _pad_edges(src1, dst1, N2, 819200)
    acc1, cnt1 = _seg_sum_sc(y1, s1p.reshape(-1, W), d1p.reshape(-1, W), 10112)

    h, z2 = _combine1(x[:N2], w1t, b1.reshape(1, -1),
                      acc1[0], acc1[1], cnt1[0], cnt1[1], w2b, 1000)

    s2p, d2p = _pad_edges(src2, dst2, N3, 163840)
    acc2, cnt2 = _seg_sum_sc(z2, s2p.reshape(-1, W), d2p.reshape(-1, W), 2176)

    out = _combine2(h[:N3], w2t, b2.reshape(1, -1),
                    acc2[0], acc2[1], cnt2[0], cnt2[1])
    return out


# speedup vs baseline: 8.8854x; 1.0331x over previous
import functools

import jax
import jax.numpy as jnp
from jax import lax
from jax.experimental import pallas as pl
from jax.experimental.pallas import tpu as pltpu
from jax.experimental.pallas import tpu_sc as plsc

F32 = jnp.float32
W = 128
NC, NS = 2, 16
NW = NC * NS


def _seg_sum_sc(y, src2d, dst2d, n_pad, n0, n1):
    D = y.shape[1]
    rpt = n_pad // NS
    WC = 8
    nw0, nw1 = n0 // WC, n1 // WC

    mesh = plsc.VectorSubcoreMesh(core_axis_name="c", subcore_axis_name="s")

    @functools.partial(
        pl.kernel,
        out_type=(jax.ShapeDtypeStruct((NC, n_pad, D), F32),
                  jax.ShapeDtypeStruct((NC, n_pad, 16), F32)),
        mesh=mesh,
        scratch_types=[
            pltpu.VMEM((2, WC, W), jnp.int32),
            pltpu.VMEM((2, WC, W), jnp.int32),
            pltpu.VMEM((2, W, D), F32),
            pltpu.VMEM((W, 16), F32),
            pltpu.VMEM_SHARED((n_pad, D), F32),
            pltpu.VMEM_SHARED((n_pad, 16), F32),
            pltpu.SemaphoreType.DMA,
            pltpu.SemaphoreType.DMA,
            pltpu.SemaphoreType.DMA,
            pltpu.SemaphoreType.DMA,
        ],
        compiler_params=pltpu.CompilerParams(use_tc_tiling_on_sc=False),
    )
    def ker(y_hbm, src_hbm, dst_hbm, z_d_hbm, z_c_hbm, ones_hbm,
            acc_out, cnt_out,
            sidx_v, didx_v, rows_v, ones_v, acc_sh, cnt_sh,
            gsem0, gsem1, wsem_s, wsem_d):
        cid = lax.axis_index("c")
        sid = lax.axis_index("s")
        row0 = sid * rpt
        n_mine = jnp.where(cid == 0, n0, n1)
        n_win = jnp.where(cid == 0, nw0, nw1)
        base = jnp.where(cid == 0, sid * n0, NS * n0 + sid * n1)

        pltpu.sync_copy(src_hbm.at[pl.ds(base, WC)], sidx_v.at[0])
        pltpu.sync_copy(dst_hbm.at[pl.ds(base, WC)], didx_v.at[0])
        pltpu.sync_copy(z_d_hbm.at[pl.ds(row0, rpt)], acc_sh.at[pl.ds(row0, rpt)])
        pltpu.sync_copy(z_c_hbm.at[pl.ds(row0, rpt)], cnt_sh.at[pl.ds(row0, rpt)])
        pltpu.sync_copy(ones_hbm, ones_v)
        plsc.subcore_barrier()

        pltpu.async_copy(y_hbm.at[sidx_v.at[0, 0]], rows_v.at[0], gsem0)
        gsems = (gsem0, gsem1)

        @pl.loop(0, n_win)
        def _(w):
            p = lax.rem(w, 2)
            nb = base + (w + 1) * WC

            @pl.when(w + 1 < n_win)
            def _():
                pltpu.async_copy(src_hbm.at[pl.ds(nb, WC)],
                                 sidx_v.at[1 - p], wsem_s)
                pltpu.async_copy(dst_hbm.at[pl.ds(nb, WC)],
                                 didx_v.at[1 - p], wsem_d)

            for j in range(WC):
                k = w * WC + j
                b = j % 2

                if j + 1 < WC:
                    pltpu.async_copy(y_hbm.at[sidx_v.at[p, j + 1]],
                                     rows_v.at[1 - b], gsems[1 - b])
                else:
                    @pl.when(w + 1 < n_win)
                    def _():
                        pltpu.make_async_copy(src_hbm.at[pl.ds(nb, WC)],
                                              sidx_v.at[1 - p], wsem_s).wait()
                        pltpu.make_async_copy(dst_hbm.at[pl.ds(nb, WC)],
                                              didx_v.at[1 - p], wsem_d).wait()
                        pltpu.async_copy(y_hbm.at[sidx_v.at[1 - p, 0]],
                                         rows_v.at[1 - b], gsems[1 - b])

                pltpu.make_async_copy(y_hbm.at[sidx_v.at[p, j]],
                                      rows_v.at[b], gsems[b]).wait()
                pltpu.sync_copy(rows_v.at[b], acc_sh.at[didx_v.at[p, j]],
                                add=True)
                pltpu.sync_copy(ones_v, cnt_sh.at[didx_v.at[p, j]],
                                add=True)

        plsc.subcore_barrier()
        pltpu.sync_copy(acc_sh.at[pl.ds(row0, rpt)],
                        acc_out.at[cid, pl.ds(row0, rpt)])
        pltpu.sync_copy(cnt_sh.at[pl.ds(row0, rpt)],
                        cnt_out.at[cid, pl.ds(row0, rpt)])

    zeros_d = jnp.zeros((n_pad, D), F32)
    zeros_c = jnp.zeros((n_pad, 16), F32)
    ones = jnp.ones((W, 16), F32)
    return ker(y, src2d, dst2d, zeros_d, zeros_c, ones)


def _pad_edges(src, dst, n_seg, e_pad):
    E = src.shape[0]
    src = jnp.concatenate([src, jnp.zeros((e_pad - E,), src.dtype)])
    dst = jnp.concatenate([dst, jnp.full((e_pad - E,), n_seg, dst.dtype)])
    return src, dst


def _mm_body(x_ref, w_ref, o_ref):
    o_ref[...] = jnp.dot(x_ref[...], w_ref[...], preferred_element_type=F32)


def _project(x, w, bm):
    M, K = x.shape
    N = w.shape[1]
    return pl.pallas_call(
        _mm_body,
        grid=(M // bm,),
        in_specs=[pl.BlockSpec((bm, K), lambda i: (i, 0)),
                  pl.BlockSpec((K, N), lambda i: (0, 0))],
        out_specs=pl.BlockSpec((bm, N), lambda i: (i, 0)),
        out_shape=jax.ShapeDtypeStruct((M, N), F32),
    )(x, w)


def _combine1_body(x_ref, wt_ref, b_ref, a0_ref, a1_ref, c0_ref, c1_ref,
                   wb2_ref, h_ref, z_ref):
    cnt = c0_ref[:, :1] + c1_ref[:, :1]
    agg = (a0_ref[...] + a1_ref[...]) / jnp.maximum(cnt, 1.0)
    h = jnp.dot(x_ref[...], wt_ref[...], preferred_element_type=F32)
    h = jnp.maximum(h + b_ref[...] + agg, 0.0)
    h_ref[...] = h
    z_ref[...] = jnp.dot(h, wb2_ref[...], preferred_element_type=F32)


def _combine1(x, wt, b, a0, a1, c0, c1, wb2, bm):
    M, K = x.shape
    N = wt.shape[1]
    return pl.pallas_call(
        _combine1_body,
        grid=(M // bm,),
        in_specs=[pl.BlockSpec((bm, K), lambda i: (i, 0)),
                  pl.BlockSpec((K, N), lambda i: (0, 0)),
                  pl.BlockSpec((1, N), lambda i: (0, 0)),
                  pl.BlockSpec((bm, N), lambda i: (i, 0)),
                  pl.BlockSpec((bm, N), lambda i: (i, 0)),
                  pl.BlockSpec((bm, 16), lambda i: (i, 0)),
                  pl.BlockSpec((bm, 16), lambda i: (i, 0)),
                  pl.BlockSpec((N, N), lambda i: (0, 0))],
        out_specs=[pl.BlockSpec((bm, N), lambda i: (i, 0)),
                   pl.BlockSpec((bm, N), lambda i: (i, 0))],
        out_shape=[jax.ShapeDtypeStruct((M, N), F32),
                   jax.ShapeDtypeStruct((M, N), F32)],
    )(x, wt, b, a0, a1, c0, c1, wb2)


def _combine2_body(h_ref, wt_ref, b_ref, a0_ref, a1_ref, c0_ref, c1_ref,
                   o_ref):
    cnt = c0_ref[:, :1] + c1_ref[:, :1]
    agg = (a0_ref[...] + a1_ref[...]) / jnp.maximum(cnt, 1.0)
    o_ref[...] = (jnp.dot(h_ref[...], wt_ref[...], preferred_element_type=F32)
                  + b_ref[...] + agg)


def _combine2(h, wt, b, a0, a1, c0, c1):
    M, K = h.shape
    N = wt.shape[1]
    return pl.pallas_call(
        _combine2_body,
        grid=(1,),
        in_specs=[pl.BlockSpec((M, K), lambda i: (0, 0)),
                  pl.BlockSpec((K, N), lambda i: (0, 0)),
                  pl.BlockSpec((1, N), lambda i: (0, 0)),
                  pl.BlockSpec((M, N), lambda i: (0, 0)),
                  pl.BlockSpec((M, N), lambda i: (0, 0)),
                  pl.BlockSpec((M, 16), lambda i: (0, 0)),
                  pl.BlockSpec((M, 16), lambda i: (0, 0))],
        out_specs=pl.BlockSpec((M, N), lambda i: (0, 0)),
        out_shape=jax.ShapeDtypeStruct((M, N), F32),
    )(h, wt, b, a0, a1, c0, c1)


def kernel(x, src1, dst1, src2, dst2, w1, b1, w2, b2):
    D = 64
    N2, N3 = 10000, 2048
    w1t, w1b = w1[:2 * D], w1[2 * D:]
    w2t, w2b = w2[:D], w2[D:]

    y1 = _project(x, w1b, 2000)
    s1p, d1p = _pad_edges(src1, dst1, N2, 819200)
    acc1, cnt1 = _seg_sum_sc(y1, s1p.reshape(-1, W), d1p.reshape(-1, W),
                             10112, 280, 120)

    h, z2 = _combine1(x[:N2], w1t, b1.reshape(1, -1),
                      acc1[0], acc1[1], cnt1[0], cnt1[1], w2b, 1000)

    s2p, d2p = _pad_edges(src2, dst2, N3, 163840)
    acc2, cnt2 = _seg_sum_sc(z2, s2p.reshape(-1, W), d2p.reshape(-1, W),
                             2176, 56, 24)

    out = _combine2(h[:N3], w2t, b2.reshape(1, -1),
                    acc2[0], acc2[1], cnt2[0], cnt2[1])
    return out


# speedup vs baseline: 8.9032x; 1.0020x over previous
import functools

import jax
import jax.numpy as jnp
from jax import lax
from jax.experimental import pallas as pl
from jax.experimental.pallas import tpu as pltpu
from jax.experimental.pallas import tpu_sc as plsc

F32 = jnp.float32
W = 128
NC, NS = 2, 16
NW = NC * NS


def _seg_sum_sc(y, src2d, dst2d, n_pad, n0, n1):
    D = y.shape[1]
    rpt = n_pad // NS
    n_max = max(n0, n1)

    mesh = plsc.VectorSubcoreMesh(core_axis_name="c", subcore_axis_name="s")

    @functools.partial(
        pl.kernel,
        out_type=(jax.ShapeDtypeStruct((NC, n_pad, D), F32),
                  jax.ShapeDtypeStruct((NC, n_pad, 16), F32)),
        mesh=mesh,
        scratch_types=[
            pltpu.VMEM((n_max, W), jnp.int32),
            pltpu.VMEM((2, W), jnp.int32),
            pltpu.VMEM((2, W, D), F32),
            pltpu.VMEM((W, 16), F32),
            pltpu.VMEM_SHARED((n_pad, D), F32),
            pltpu.VMEM_SHARED((n_pad, 16), F32),
            pltpu.SemaphoreType.DMA,
            pltpu.SemaphoreType.DMA,
            pltpu.SemaphoreType.DMA,
            pltpu.SemaphoreType.DMA,
        ],
        compiler_params=pltpu.CompilerParams(use_tc_tiling_on_sc=False),
    )
    def ker(y_hbm, src_hbm, dst_hbm, z_d_hbm, z_c_hbm, ones_hbm,
            acc_out, cnt_out,
            sidx_v, didx_v, rows_v, ones_v, acc_sh, cnt_sh,
            gsem0, gsem1, dsem0, dsem1):
        cid = lax.axis_index("c")
        sid = lax.axis_index("s")
        row0 = sid * rpt
        gsems = (gsem0, gsem1)
        dsems = (dsem0, dsem1)

        pltpu.sync_copy(z_d_hbm.at[pl.ds(row0, rpt)], acc_sh.at[pl.ds(row0, rpt)])
        pltpu.sync_copy(z_c_hbm.at[pl.ds(row0, rpt)], cnt_sh.at[pl.ds(row0, rpt)])
        pltpu.sync_copy(ones_hbm, ones_v)
        plsc.subcore_barrier()

        def pipeline(n_chunks, chunk0):
            pltpu.sync_copy(src_hbm.at[pl.ds(chunk0, n_chunks)],
                            sidx_v.at[pl.ds(0, n_chunks)])
            pltpu.async_copy(dst_hbm.at[chunk0], didx_v.at[0], dsem0)
            pltpu.async_copy(y_hbm.at[sidx_v.at[0]], rows_v.at[0], gsem0)

            @pl.loop(0, n_chunks, step=2)
            def _(k0):
                for b in range(2):
                    k = k0 + b

                    @pl.when(k + 1 < n_chunks)
                    def _():
                        pltpu.async_copy(dst_hbm.at[chunk0 + k + 1],
                                         didx_v.at[1 - b], dsems[1 - b])
                        pltpu.async_copy(y_hbm.at[sidx_v.at[k + 1]],
                                         rows_v.at[1 - b], gsems[1 - b])

                    pltpu.make_async_copy(y_hbm.at[sidx_v.at[k]],
                                          rows_v.at[b], gsems[b]).wait()
                    pltpu.make_async_copy(dst_hbm.at[chunk0 + k],
                                          didx_v.at[b], dsems[b]).wait()
                    pltpu.sync_copy(rows_v.at[b], acc_sh.at[didx_v.at[b]],
                                    add=True)
                    pltpu.sync_copy(ones_v, cnt_sh.at[didx_v.at[b]],
                                    add=True)

        @pl.when(cid == 0)
        def _():
            pipeline(n0, sid * n0)

        @pl.when(cid == 1)
        def _():
            pipeline(n1, NS * n0 + sid * n1)

        plsc.subcore_barrier()
        pltpu.sync_copy(acc_sh.at[pl.ds(row0, rpt)],
                        acc_out.at[cid, pl.ds(row0, rpt)])
        pltpu.sync_copy(cnt_sh.at[pl.ds(row0, rpt)],
                        cnt_out.at[cid, pl.ds(row0, rpt)])

    zeros_d = jnp.zeros((n_pad, D), F32)
    zeros_c = jnp.zeros((n_pad, 16), F32)
    ones = jnp.ones((W, 16), F32)
    return ker(y, src2d, dst2d, zeros_d, zeros_c, ones)


def _pad_edges(src, dst, n_seg, e_pad):
    E = src.shape[0]
    src = jnp.concatenate([src, jnp.zeros((e_pad - E,), src.dtype)])
    dst = jnp.concatenate([dst, jnp.full((e_pad - E,), n_seg, dst.dtype)])
    return src, dst


def _mm_body(x_ref, w_ref, o_ref):
    o_ref[...] = jnp.dot(x_ref[...], w_ref[...], preferred_element_type=F32)


def _project(x, w, bm):
    M, K = x.shape
    N = w.shape[1]
    return pl.pallas_call(
        _mm_body,
        grid=(M // bm,),
        in_specs=[pl.BlockSpec((bm, K), lambda i: (i, 0)),
                  pl.BlockSpec((K, N), lambda i: (0, 0))],
        out_specs=pl.BlockSpec((bm, N), lambda i: (i, 0)),
        out_shape=jax.ShapeDtypeStruct((M, N), F32),
    )(x, w)


def _combine1_body(x_ref, wt_ref, b_ref, a0_ref, a1_ref, c0_ref, c1_ref,
                   wb2_ref, h_ref, z_ref):
    cnt = c0_ref[:, :1] + c1_ref[:, :1]
    agg = (a0_ref[...] + a1_ref[...]) / jnp.maximum(cnt, 1.0)
    h = jnp.dot(x_ref[...], wt_ref[...], preferred_element_type=F32)
    h = jnp.maximum(h + b_ref[...] + agg, 0.0)
    h_ref[...] = h
    z_ref[...] = jnp.dot(h, wb2_ref[...], preferred_element_type=F32)


def _combine1(x, wt, b, a0, a1, c0, c1, wb2, bm):
    M, K = x.shape
    N = wt.shape[1]
    return pl.pallas_call(
        _combine1_body,
        grid=(M // bm,),
        in_specs=[pl.BlockSpec((bm, K), lambda i: (i, 0)),
                  pl.BlockSpec((K, N), lambda i: (0, 0)),
                  pl.BlockSpec((1, N), lambda i: (0, 0)),
                  pl.BlockSpec((bm, N), lambda i: (i, 0)),
                  pl.BlockSpec((bm, N), lambda i: (i, 0)),
                  pl.BlockSpec((bm, 16), lambda i: (i, 0)),
                  pl.BlockSpec((bm, 16), lambda i: (i, 0)),
                  pl.BlockSpec((N, N), lambda i: (0, 0))],
        out_specs=[pl.BlockSpec((bm, N), lambda i: (i, 0)),
                   pl.BlockSpec((bm, N), lambda i: (i, 0))],
        out_shape=[jax.ShapeDtypeStruct((M, N), F32),
                   jax.ShapeDtypeStruct((M, N), F32)],
    )(x, wt, b, a0, a1, c0, c1, wb2)


def _combine2_body(h_ref, wt_ref, b_ref, a0_ref, a1_ref, c0_ref, c1_ref,
                   o_ref):
    cnt = c0_ref[:, :1] + c1_ref[:, :1]
    agg = (a0_ref[...] + a1_ref[...]) / jnp.maximum(cnt, 1.0)
    o_ref[...] = (jnp.dot(h_ref[...], wt_ref[...], preferred_element_type=F32)
                  + b_ref[...] + agg)


def _combine2(h, wt, b, a0, a1, c0, c1):
    M, K = h.shape
    N = wt.shape[1]
    return pl.pallas_call(
        _combine2_body,
        grid=(1,),
        in_specs=[pl.BlockSpec((M, K), lambda i: (0, 0)),
                  pl.BlockSpec((K, N), lambda i: (0, 0)),
                  pl.BlockSpec((1, N), lambda i: (0, 0)),
                  pl.BlockSpec((M, N), lambda i: (0, 0)),
                  pl.BlockSpec((M, N), lambda i: (0, 0)),
                  pl.BlockSpec((M, 16), lambda i: (0, 0)),
                  pl.BlockSpec((M, 16), lambda i: (0, 0))],
        out_specs=pl.BlockSpec((M, N), lambda i: (0, 0)),
        out_shape=jax.ShapeDtypeStruct((M, N), F32),
    )(h, wt, b, a0, a1, c0, c1)


def kernel(x, src1, dst1, src2, dst2, w1, b1, w2, b2):
    D = 64
    N2, N3 = 10000, 2048
    w1t, w1b = w1[:2 * D], w1[2 * D:]
    w2t, w2b = w2[:D], w2[D:]

    y1 = _project(x, w1b, 2000)
    s1p, d1p = _pad_edges(src1, dst1, N2, 819200)
    acc1, cnt1 = _seg_sum_sc(y1, s1p.reshape(-1, W), d1p.reshape(-1, W),
                             10112, 280, 120)

    h, z2 = _combine1(x[:N2], w1t, b1.reshape(1, -1),
                      acc1[0], acc1[1], cnt1[0], cnt1[1], w2b, 1000)

    s2p, d2p = _pad_edges(src2, dst2, N3, 163840)
    acc2, cnt2 = _seg_sum_sc(z2, s2p.reshape(-1, W), d2p.reshape(-1, W),
                             2176, 56, 24)

    out = _combine2(h[:N3], w2t, b2.reshape(1, -1),
                    acc2[0], acc2[1], cnt2[0], cnt2[1])
    return out
